# baseline (speedup 1.0000x reference)
import jax
import jax.numpy as jnp
from jax.experimental import pallas as pl
from jax.experimental.pallas import tpu as pltpu

NUM_CLASSES = 1000
B0 = 4096
B1 = 20
CHUNK = 32
N_CHUNKS = B0 // CHUNK
S = 8


def _one_hot_kernel(idx_ref, out_ref, scratch_ref, sem_ref):
    def chunk_copy(i, s):
        return pltpu.make_async_copy(
            scratch_ref.at[s],
            out_ref.at[pl.ds(i * CHUNK, CHUNK)],
            sem_ref.at[s],
        )

    def body(i, carry):
        s = jax.lax.rem(i, S)

        @pl.when(i >= S)
        def _wait_prev():
            chunk_copy(i - S, s).wait()

        idx = idx_ref[pl.ds(i * CHUNK, CHUNK), :]
        classes = jax.lax.broadcasted_iota(
            jnp.int32, (CHUNK, B1, NUM_CLASSES), 2)
        scratch_ref[s] = (idx[:, :, None] == classes).astype(jnp.float32)
        chunk_copy(i, s).start()
        return carry

    jax.lax.fori_loop(0, N_CHUNKS, body, 0)
    for j in range(S):
        i = N_CHUNKS - S + j
        chunk_copy(i, i % S).wait()


def kernel(inputs):
    idx = inputs.astype(jnp.int32)
    out = pl.pallas_call(
        _one_hot_kernel,
        in_specs=[pl.BlockSpec((B0, B1), lambda: (0, 0))],
        out_specs=pl.BlockSpec(memory_space=pl.ANY),
        out_shape=jax.ShapeDtypeStruct((B0, B1, NUM_CLASSES), jnp.float32),
        scratch_shapes=[
            pltpu.VMEM((S, CHUNK, B1, NUM_CLASSES), jnp.float32),
            pltpu.SemaphoreType.DMA((S,)),
        ],
    )(idx)
    return out


# speedup vs baseline: 1.0077x; 1.0077x over previous
import jax
import jax.numpy as jnp
from jax.experimental import pallas as pl
from jax.experimental.pallas import tpu as pltpu

NUM_CLASSES = 1000
B0 = 4096
B1 = 20
CHUNK = 32
N_CHUNKS = B0 // CHUNK
S = 8


def _one_hot_kernel(idx_ref, out_ref, scratch_ref, sem_ref):
    def chunk_copy(i, s):
        return pltpu.make_async_copy(
            scratch_ref.at[s],
            out_ref.at[pl.ds(i * CHUNK, CHUNK)],
            sem_ref.at[s],
        )

    for i in range(N_CHUNKS):
        s = i % S
        if i >= S:
            chunk_copy(i - S, s).wait()
        idx = idx_ref[pl.ds(i * CHUNK, CHUNK), :]
        classes = jax.lax.broadcasted_iota(
            jnp.int32, (CHUNK, B1, NUM_CLASSES), 2)
        scratch_ref[s] = (idx[:, :, None] == classes).astype(jnp.float32)
        chunk_copy(i, s).start()
    for j in range(S):
        i = N_CHUNKS - S + j
        chunk_copy(i, i % S).wait()


def kernel(inputs):
    idx = inputs.astype(jnp.int32)
    out = pl.pallas_call(
        _one_hot_kernel,
        in_specs=[pl.BlockSpec((B0, B1), lambda: (0, 0))],
        out_specs=pl.BlockSpec(memory_space=pl.ANY),
        out_shape=jax.ShapeDtypeStruct((B0, B1, NUM_CLASSES), jnp.float32),
        scratch_shapes=[
            pltpu.VMEM((S, CHUNK, B1, NUM_CLASSES), jnp.float32),
            pltpu.SemaphoreType.DMA((S,)),
        ],
    )(idx)
    return out
